# baseline (speedup 1.0000x reference)
import jax
import jax.numpy as jnp
from jax import lax
from jax.experimental import pallas as pl

BLANK = 0
DEPTH = 1000
ROWS_PER_BLOCK = 1024


def _onehot_block(idx_ref, out_ref):
    idx = idx_ref[0]
    shifted = jnp.where(idx == BLANK, -1, idx)
    iota = lax.broadcasted_iota(jnp.int32, out_ref.shape, 1)
    out_ref[...] = (shifted == iota).astype(jnp.float32)


def kernel(outputs, outputs_length):
    b, t = outputs.shape
    n = b * t
    r = ROWS_PER_BLOCK
    g = n // r
    idx3 = outputs.astype(jnp.int32).reshape(g, r, 1)
    flat = pl.pallas_call(
        _onehot_block,
        grid=(g,),
        in_specs=[pl.BlockSpec((1, r, 1), lambda i: (i, 0, 0))],
        out_specs=pl.BlockSpec((r, DEPTH), lambda i: (i, 0)),
        out_shape=jax.ShapeDtypeStruct((n, DEPTH), jnp.float32),
    )(idx3)
    return (flat.reshape(b, t, DEPTH), outputs_length)


# speedup vs baseline: 1.3364x; 1.3364x over previous
import jax
import jax.numpy as jnp
from jax import lax
from jax.experimental import pallas as pl

BLANK = 0
DEPTH = 1000
BATCH_BLOCK = 32


def _onehot_block(idx_ref, out_ref):
    idx = idx_ref[...]
    shifted = jnp.where(idx == BLANK, -1, idx)
    iota = lax.broadcasted_iota(jnp.int32, out_ref.shape, 2)
    out_ref[...] = (shifted == iota).astype(jnp.float32)


def kernel(outputs, outputs_length):
    b, t = outputs.shape
    bb = BATCH_BLOCK
    idx3 = outputs.astype(jnp.int32).reshape(b, t, 1)
    one_hot = pl.pallas_call(
        _onehot_block,
        grid=(b // bb,),
        in_specs=[pl.BlockSpec((bb, t, 1), lambda i: (i, 0, 0))],
        out_specs=pl.BlockSpec((bb, t, DEPTH), lambda i: (i, 0, 0)),
        out_shape=jax.ShapeDtypeStruct((b, t, DEPTH), jnp.float32),
    )(idx3)
    return (one_hot, outputs_length)


# speedup vs baseline: 1.3431x; 1.0051x over previous
import jax
import jax.numpy as jnp
from jax import lax
from jax.experimental import pallas as pl

BLANK = 0
DEPTH = 1000
BATCH_BLOCK = 64


def _onehot_block(idx_ref, out_ref):
    idx = idx_ref[...]
    shifted = jnp.where(idx == BLANK, -1, idx)
    iota = lax.broadcasted_iota(jnp.int32, out_ref.shape, 2)
    out_ref[...] = (shifted == iota).astype(jnp.float32)


def kernel(outputs, outputs_length):
    b, t = outputs.shape
    bb = BATCH_BLOCK
    idx3 = outputs.astype(jnp.int32).reshape(b, t, 1)
    one_hot = pl.pallas_call(
        _onehot_block,
        grid=(b // bb,),
        in_specs=[pl.BlockSpec((bb, t, 1), lambda i: (i, 0, 0))],
        out_specs=pl.BlockSpec((bb, t, DEPTH), lambda i: (i, 0, 0)),
        out_shape=jax.ShapeDtypeStruct((b, t, DEPTH), jnp.float32),
    )(idx3)
    return (one_hot, outputs_length)


# speedup vs baseline: 1.3513x; 1.0061x over previous
import jax
import jax.numpy as jnp
from jax import lax
from jax.experimental import pallas as pl
from jax.experimental.pallas import tpu as pltpu

BLANK = 0
DEPTH = 1000
SUB_BATCH = 16
NUM_SLOTS = 8
STEP_BATCH = SUB_BATCH * NUM_SLOTS


def _onehot_body(idx_ref, out_ref, scratch, sems):
    i = pl.program_id(0)
    for k in range(NUM_SLOTS):
        rows = pl.ds(k * SUB_BATCH, SUB_BATCH)

        @pl.when(i > 0)
        def _wait_prev():
            pltpu.make_async_copy(
                scratch.at[rows],
                out_ref.at[pl.ds(((i - 1) * NUM_SLOTS + k) * SUB_BATCH, SUB_BATCH)],
                sems.at[k],
            ).wait()

        idx = idx_ref[rows]
        shifted = jnp.where(idx == BLANK, -1, idx)
        iota = lax.broadcasted_iota(
            jnp.int32, (SUB_BATCH, idx_ref.shape[1], DEPTH), 2)
        scratch[rows] = (shifted == iota).astype(jnp.float32)

        pltpu.make_async_copy(
            scratch.at[rows],
            out_ref.at[pl.ds((i * NUM_SLOTS + k) * SUB_BATCH, SUB_BATCH)],
            sems.at[k],
        ).start()

    @pl.when(i == pl.num_programs(0) - 1)
    def _drain():
        for k in range(NUM_SLOTS):
            rows = pl.ds(k * SUB_BATCH, SUB_BATCH)
            pltpu.make_async_copy(
                scratch.at[rows],
                out_ref.at[pl.ds((i * NUM_SLOTS + k) * SUB_BATCH, SUB_BATCH)],
                sems.at[k],
            ).wait()


def kernel(outputs, outputs_length):
    b, t = outputs.shape
    idx3 = outputs.astype(jnp.int32).reshape(b, t, 1)
    one_hot = pl.pallas_call(
        _onehot_body,
        grid=(b // STEP_BATCH,),
        in_specs=[pl.BlockSpec((STEP_BATCH, t, 1), lambda i: (i, 0, 0))],
        out_specs=pl.BlockSpec(memory_space=pl.ANY),
        out_shape=jax.ShapeDtypeStruct((b, t, DEPTH), jnp.float32),
        scratch_shapes=[
            pltpu.VMEM((STEP_BATCH, t, DEPTH), jnp.float32),
            pltpu.SemaphoreType.DMA((NUM_SLOTS,)),
        ],
    )(idx3)
    return (one_hot, outputs_length)


# speedup vs baseline: 6.3552x; 4.7030x over previous
import jax
import jax.numpy as jnp
from jax import lax
from jax.experimental import pallas as pl
from jax.experimental.pallas import tpu as pltpu

BLANK = 0
DEPTH = 1000
NUM_SLOTS = 5


def _onehot_body(idx_ref, out_ref, scratch, sems):
    i = pl.program_id(0)
    for k in range(NUM_SLOTS):
        @pl.when(i > 0)
        def _wait_prev():
            pltpu.make_async_copy(
                scratch.at[k],
                out_ref.at[(i - 1) * NUM_SLOTS + k],
                sems.at[k],
            ).wait()

        row = idx_ref[k]
        shifted = jnp.where(row == BLANK, -1, row)
        iota = lax.broadcasted_iota(
            jnp.int32, (DEPTH, idx_ref.shape[2]), 0)
        scratch[k] = (shifted == iota).astype(jnp.float32)

        pltpu.make_async_copy(
            scratch.at[k],
            out_ref.at[i * NUM_SLOTS + k],
            sems.at[k],
        ).start()

    @pl.when(i == pl.num_programs(0) - 1)
    def _drain():
        for k in range(NUM_SLOTS):
            pltpu.make_async_copy(
                scratch.at[k],
                out_ref.at[i * NUM_SLOTS + k],
                sems.at[k],
            ).wait()


def kernel(outputs, outputs_length):
    b, t = outputs.shape
    idx3 = outputs.astype(jnp.int32).T.reshape(t, 1, b)
    one_hot_t = pl.pallas_call(
        _onehot_body,
        grid=(t // NUM_SLOTS,),
        in_specs=[pl.BlockSpec((NUM_SLOTS, 1, b), lambda i: (i, 0, 0))],
        out_specs=pl.BlockSpec(memory_space=pl.ANY),
        out_shape=jax.ShapeDtypeStruct((t, DEPTH, b), jnp.float32),
        scratch_shapes=[
            pltpu.VMEM((NUM_SLOTS, DEPTH, b), jnp.float32),
            pltpu.SemaphoreType.DMA((NUM_SLOTS,)),
        ],
    )(idx3)
    return (jnp.transpose(one_hot_t, (2, 0, 1)), outputs_length)
